# Initial kernel scaffold; baseline (speedup 1.0000x reference)
import jax
import jax.numpy as jnp
from jax.experimental import pallas as pl


def kernel(node_feats, edge_index, edge_geo, cond_feats, batch_ids, W_nproj, b_nproj, W_egeo, b_egeo, W_cond, b_cond, W_x):
    raise NotImplementedError("write your pallas kernel here")


# speedup vs baseline: 3.8205x; 3.8205x over previous
import functools

import jax
import jax.numpy as jnp
from jax import lax
from jax.experimental import pallas as pl
from jax.experimental.pallas import tpu as pltpu
from jax.experimental.pallas import tpu_sc as plsc

_N = 10000
_E = 320000
_D = 128
_GEO = 16
_B = 16


def _prep_body(node_ref, wn_ref, bn_ref, cond_ref, wc_ref, bc_ref,
               nf_ref, g_ref, b_ref):
    nf_ref[...] = (jnp.dot(node_ref[...], wn_ref[...],
                           preferred_element_type=jnp.float32) + bn_ref[...])
    gb = (jnp.dot(cond_ref[...], wc_ref[...],
                  preferred_element_type=jnp.float32) + bc_ref[...])
    g_ref[...] = gb[:, :_D] + 1.0
    b_ref[...] = gb[:, _D:]


def _prep(node_feats, W_nproj, b_nproj, cond_feats, W_cond, b_cond):
    return pl.pallas_call(
        _prep_body,
        out_shape=[
            jax.ShapeDtypeStruct((_N, _D), jnp.float32),
            jax.ShapeDtypeStruct((_B, _D), jnp.float32),
            jax.ShapeDtypeStruct((_B, _D), jnp.float32),
        ],
    )(node_feats, W_nproj, b_nproj.reshape(1, _D),
      cond_feats, W_cond, b_cond.reshape(1, 2 * _D))


def _make_gather_mul():
    info = plsc.get_sparse_core_info()
    nc, ns = info.num_cores, info.num_subcores
    nw = nc * ns
    per_w = _E // nw
    ch = 80
    nch = per_w // ch

    mesh = plsc.VectorSubcoreMesh(core_axis_name="c", subcore_axis_name="s")

    @functools.partial(
        pl.kernel, mesh=mesh,
        out_type=jax.ShapeDtypeStruct((_E, _D), jnp.float32),
        scratch_types=[
            pltpu.VMEM((ch,), jnp.int32),
            pltpu.VMEM((ch,), jnp.int32),
            pltpu.VMEM((ch, _D), jnp.float32),
            pltpu.VMEM((ch, _D), jnp.float32),
            pltpu.SemaphoreType.DMA,
            pltpu.SemaphoreType.DMA,
        ],
    )
    def gather_mul(nf_hbm, src_hbm, dst_hbm, out_hbm,
                   sidx, didx, srows, drows, sem_a, sem_b):
        wid = lax.axis_index("s") * nc + lax.axis_index("c")
        base = wid * per_w

        def chunk(k, carry):
            off = pl.multiple_of(base + k * ch, 8)
            pltpu.sync_copy(src_hbm.at[pl.ds(off, ch)], sidx)
            pltpu.sync_copy(dst_hbm.at[pl.ds(off, ch)], didx)
            cp_a = pltpu.async_copy(nf_hbm.at[sidx], srows, sem_a)
            cp_b = pltpu.async_copy(nf_hbm.at[didx], drows, sem_b)
            cp_a.wait()
            cp_b.wait()

            def row(r, c2):
                for j in range(_D // 16):
                    sl = pl.ds(j * 16, 16)
                    srows[r, sl] = srows[r, sl] * drows[r, sl]
                return c2

            lax.fori_loop(0, ch, row, 0)
            pltpu.sync_copy(srows, out_hbm.at[pl.ds(off, ch)])
            return carry

        lax.fori_loop(0, nch, chunk, 0)

    return gather_mul


def _edge_body(nj_ref, geo_ref, bid_ref, wxt_ref, weg_ref, beg_ref, wxb_ref,
               g_ref, b_ref, out_ref):
    e = (jnp.dot(geo_ref[...], weg_ref[...],
                 preferred_element_type=jnp.float32) + beg_ref[...])
    x = (jnp.dot(nj_ref[...], wxt_ref[...],
                 preferred_element_type=jnp.float32)
         + jnp.dot(e, wxb_ref[...], preferred_element_type=jnp.float32))
    tile = nj_ref.shape[0]
    oh = (bid_ref[...] == lax.broadcasted_iota(jnp.int32, (tile, _B), 1)
          ).astype(jnp.float32)
    gamma = jnp.dot(oh, g_ref[...], preferred_element_type=jnp.float32)
    beta = jnp.dot(oh, b_ref[...], preferred_element_type=jnp.float32)
    out_ref[...] = jnp.maximum(x * gamma + beta, 0.0)


def _edge_stage(n_join, edge_geo, bid2d, wxt, W_egeo, beg2d, wxb, gamma1, beta):
    tile = 1280
    grid = (_E // tile,)
    full = lambda i: (0, 0)
    return pl.pallas_call(
        _edge_body,
        grid=grid,
        in_specs=[
            pl.BlockSpec((tile, _D), lambda i: (i, 0)),
            pl.BlockSpec((tile, _GEO), lambda i: (i, 0)),
            pl.BlockSpec((tile, 1), lambda i: (i, 0)),
            pl.BlockSpec((_D, _D), full),
            pl.BlockSpec((_GEO, _GEO), full),
            pl.BlockSpec((1, _GEO), full),
            pl.BlockSpec((_GEO, _D), full),
            pl.BlockSpec((_B, _D), full),
            pl.BlockSpec((_B, _D), full),
        ],
        out_specs=pl.BlockSpec((tile, _D), lambda i: (i, 0)),
        out_shape=jax.ShapeDtypeStruct((_E, _D), jnp.float32),
        compiler_params=pltpu.CompilerParams(
            dimension_semantics=("arbitrary",)),
    )(n_join, edge_geo, bid2d, wxt, W_egeo, beg2d, wxb, gamma1, beta)


_gather_mul = _make_gather_mul()


def kernel(node_feats, edge_index, edge_geo, cond_feats, batch_ids,
           W_nproj, b_nproj, W_egeo, b_egeo, W_cond, b_cond, W_x):
    nf, gamma1, beta = _prep(node_feats, W_nproj, b_nproj,
                             cond_feats, W_cond, b_cond)
    src = edge_index[0].astype(jnp.int32)
    dst = edge_index[1].astype(jnp.int32)
    n_join = _gather_mul(nf, src, dst)
    bid2d = batch_ids.astype(jnp.int32).reshape(_E, 1)
    return _edge_stage(n_join, edge_geo, bid2d,
                       W_x[:_D], W_egeo, b_egeo.reshape(1, _GEO),
                       W_x[_D:], gamma1, beta)


# speedup vs baseline: 4.8271x; 1.2635x over previous
import functools

import jax
import jax.numpy as jnp
from jax import lax
from jax.experimental import pallas as pl
from jax.experimental.pallas import tpu as pltpu
from jax.experimental.pallas import tpu_sc as plsc

_N = 10000
_E = 320000
_D = 128
_GEO = 16
_B = 16


def _prep_body(node_ref, wn_ref, bn_ref, cond_ref, wc_ref, bc_ref,
               nf_ref, g_ref, b_ref):
    nf_ref[...] = (jnp.dot(node_ref[...], wn_ref[...],
                           preferred_element_type=jnp.float32) + bn_ref[...])
    gb = (jnp.dot(cond_ref[...], wc_ref[...],
                  preferred_element_type=jnp.float32) + bc_ref[...])
    g_ref[...] = gb[:, :_D] + 1.0
    b_ref[...] = gb[:, _D:]


def _prep(node_feats, W_nproj, b_nproj, cond_feats, W_cond, b_cond):
    return pl.pallas_call(
        _prep_body,
        out_shape=[
            jax.ShapeDtypeStruct((_N, _D), jnp.float32),
            jax.ShapeDtypeStruct((_B, _D), jnp.float32),
            jax.ShapeDtypeStruct((_B, _D), jnp.float32),
        ],
    )(node_feats, W_nproj, b_nproj.reshape(1, _D),
      cond_feats, W_cond, b_cond.reshape(1, 2 * _D))


_CH = 80


def _make_gather_mul():
    info = plsc.get_sparse_core_info()
    nc, ns = info.num_cores, info.num_subcores
    nw = nc * ns
    per_w = _E // nw
    nch = per_w // _CH

    mesh = plsc.VectorSubcoreMesh(core_axis_name="c", subcore_axis_name="s")

    @functools.partial(
        pl.kernel, mesh=mesh,
        out_type=jax.ShapeDtypeStruct((_E, _D), jnp.float32),
        scratch_types=[
            pltpu.VMEM((nch, _CH), jnp.int32),
            pltpu.VMEM((nch, _CH), jnp.int32),
            pltpu.VMEM((_CH, _D), jnp.float32),
            pltpu.VMEM((_CH, _D), jnp.float32),
            pltpu.VMEM((_CH, _D), jnp.float32),
            pltpu.VMEM((_CH, _D), jnp.float32),
            pltpu.VMEM((_CH, _D), jnp.float32),
            pltpu.VMEM((_CH, _D), jnp.float32),
            pltpu.SemaphoreType.DMA,
            pltpu.SemaphoreType.DMA,
            pltpu.SemaphoreType.DMA,
            pltpu.SemaphoreType.DMA,
            pltpu.SemaphoreType.DMA,
            pltpu.SemaphoreType.DMA,
        ],
    )
    def gather_mul(nf_hbm, src2_hbm, dst2_hbm, out_hbm,
                   sidx, didx, s0, s1, d0, d1, o0, o1,
                   ga0, ga1, gd0, gd1, wb0, wb1):
        srows = (s0, s1)
        drows = (d0, d1)
        orows = (o0, o1)
        gsem_s = (ga0, ga1)
        gsem_d = (gd0, gd1)
        wsem = (wb0, wb1)
        wid = lax.axis_index("s") * nc + lax.axis_index("c")
        ebase = wid * per_w

        def out_slice(i):
            return pl.ds(pl.multiple_of(ebase + i * _CH, 8), _CH)

        pltpu.sync_copy(src2_hbm.at[wid], sidx)
        pltpu.sync_copy(dst2_hbm.at[wid], didx)

        def start_gather(i, b):
            pltpu.async_copy(nf_hbm.at[sidx.at[i]], srows[b], gsem_s[b])
            pltpu.async_copy(nf_hbm.at[didx.at[i]], drows[b], gsem_d[b])

        def wait_gather(i, b):
            pltpu.make_async_copy(nf_hbm.at[sidx.at[i]], srows[b],
                                  gsem_s[b]).wait()
            pltpu.make_async_copy(nf_hbm.at[didx.at[i]], drows[b],
                                  gsem_d[b]).wait()

        def do_mul(b):
            sb, db, ob = srows[b], drows[b], orows[b]

            @plsc.parallel_loop(0, _CH, 1, unroll=2)
            def _(r):
                for j in range(_D // 16):
                    sl = pl.ds(j * 16, 16)
                    ob[r, sl] = sb[r, sl] * db[r, sl]

        def start_wb(i, b):
            pltpu.async_copy(orows[b], out_hbm.at[out_slice(i)], wsem[b])

        def wait_wb(i, b):
            pltpu.make_async_copy(orows[b], out_hbm.at[out_slice(i)],
                                  wsem[b]).wait()

        start_gather(0, 0)
        start_gather(1, 1)

        def pair(p, carry):
            i0 = p * 2
            for b in (0, 1):
                i = i0 + b
                wait_gather(i, b)

                @pl.when(p > 0)
                def _():
                    wait_wb(i - 2, b)

                do_mul(b)
                start_wb(i, b)

                @pl.when(i + 2 < nch)
                def _():
                    start_gather(i + 2, b)
            return carry

        lax.fori_loop(0, nch // 2, pair, 0)

        i = nch - 1
        wait_gather(i, 0)
        wait_wb(i - 2, 0)
        do_mul(0)
        start_wb(i, 0)
        wait_wb(nch - 2, 1)
        wait_wb(nch - 1, 0)

    return gather_mul


def _edge_body(nj_ref, geo_ref, bid_ref, wxt_ref, weg_ref, beg_ref, wxb_ref,
               g_ref, b_ref, out_ref):
    e = (jnp.dot(geo_ref[...], weg_ref[...],
                 preferred_element_type=jnp.float32) + beg_ref[...])
    x = (jnp.dot(nj_ref[...], wxt_ref[...],
                 preferred_element_type=jnp.float32)
         + jnp.dot(e, wxb_ref[...], preferred_element_type=jnp.float32))
    tile = nj_ref.shape[0]
    oh = (bid_ref[...] == lax.broadcasted_iota(jnp.int32, (tile, _B), 1)
          ).astype(jnp.float32)
    gamma = jnp.dot(oh, g_ref[...], preferred_element_type=jnp.float32)
    beta = jnp.dot(oh, b_ref[...], preferred_element_type=jnp.float32)
    out_ref[...] = jnp.maximum(x * gamma + beta, 0.0)


def _edge_stage(n_join, edge_geo, bid2d, wxt, W_egeo, beg2d, wxb, gamma1, beta):
    tile = 1280
    grid = (_E // tile,)
    full = lambda i: (0, 0)
    return pl.pallas_call(
        _edge_body,
        grid=grid,
        in_specs=[
            pl.BlockSpec((tile, _D), lambda i: (i, 0)),
            pl.BlockSpec((tile, _GEO), lambda i: (i, 0)),
            pl.BlockSpec((tile, 1), lambda i: (i, 0)),
            pl.BlockSpec((_D, _D), full),
            pl.BlockSpec((_GEO, _GEO), full),
            pl.BlockSpec((1, _GEO), full),
            pl.BlockSpec((_GEO, _D), full),
            pl.BlockSpec((_B, _D), full),
            pl.BlockSpec((_B, _D), full),
        ],
        out_specs=pl.BlockSpec((tile, _D), lambda i: (i, 0)),
        out_shape=jax.ShapeDtypeStruct((_E, _D), jnp.float32),
        compiler_params=pltpu.CompilerParams(
            dimension_semantics=("arbitrary",)),
    )(n_join, edge_geo, bid2d, wxt, W_egeo, beg2d, wxb, gamma1, beta)


_gather_mul = _make_gather_mul()


def kernel(node_feats, edge_index, edge_geo, cond_feats, batch_ids,
           W_nproj, b_nproj, W_egeo, b_egeo, W_cond, b_cond, W_x):
    nf, gamma1, beta = _prep(node_feats, W_nproj, b_nproj,
                             cond_feats, W_cond, b_cond)
    src2 = edge_index[0].astype(jnp.int32).reshape(32, _E // (32 * _CH), _CH)
    dst2 = edge_index[1].astype(jnp.int32).reshape(32, _E // (32 * _CH), _CH)
    n_join = _gather_mul(nf, src2, dst2)
    bid2d = batch_ids.astype(jnp.int32).reshape(_E, 1)
    return _edge_stage(n_join, edge_geo, bid2d,
                       W_x[:_D], W_egeo, b_egeo.reshape(1, _GEO),
                       W_x[_D:], gamma1, beta)


# speedup vs baseline: 6.4248x; 1.3310x over previous
import functools

import jax
import jax.numpy as jnp
from jax import lax
from jax.experimental import pallas as pl
from jax.experimental.pallas import tpu as pltpu
from jax.experimental.pallas import tpu_sc as plsc

_N = 10000
_E = 320000
_D = 128
_GEO = 16
_B = 16


def _prep_body(node_ref, wn_ref, bn_ref, cond_ref, wc_ref, bc_ref,
               weg_ref, beg_ref, wxb_ref,
               nf_ref, g_ref, b_ref, wg_ref, bg_ref):
    nf_ref[...] = (jnp.dot(node_ref[...], wn_ref[...],
                           preferred_element_type=jnp.float32) + bn_ref[...])
    gb = (jnp.dot(cond_ref[...], wc_ref[...],
                  preferred_element_type=jnp.float32) + bc_ref[...])
    g_ref[...] = gb[:, :_D] + 1.0
    b_ref[...] = gb[:, _D:]
    wg_ref[...] = jnp.dot(weg_ref[...], wxb_ref[...],
                          preferred_element_type=jnp.float32)
    bg_ref[...] = jnp.dot(beg_ref[...], wxb_ref[...],
                          preferred_element_type=jnp.float32)


def _prep(node_feats, W_nproj, b_nproj, cond_feats, W_cond, b_cond,
          W_egeo, b_egeo, wxb):
    return pl.pallas_call(
        _prep_body,
        out_shape=[
            jax.ShapeDtypeStruct((_N, _D), jnp.float32),
            jax.ShapeDtypeStruct((_B, _D), jnp.float32),
            jax.ShapeDtypeStruct((_B, _D), jnp.float32),
            jax.ShapeDtypeStruct((_GEO, _D), jnp.float32),
            jax.ShapeDtypeStruct((1, _D), jnp.float32),
        ],
    )(node_feats, W_nproj, b_nproj.reshape(1, _D),
      cond_feats, W_cond, b_cond.reshape(1, 2 * _D),
      W_egeo, b_egeo.reshape(1, _GEO), wxb)


_CH = 80


def _make_gather_mul():
    info = plsc.get_sparse_core_info()
    nc, ns = info.num_cores, info.num_subcores
    nw = nc * ns
    per_w = _E // nw
    nch = per_w // _CH

    mesh = plsc.VectorSubcoreMesh(core_axis_name="c", subcore_axis_name="s")

    @functools.partial(
        pl.kernel, mesh=mesh,
        out_type=jax.ShapeDtypeStruct((_E, _D), jnp.float32),
        scratch_types=[
            pltpu.VMEM((nch, _CH), jnp.int32),
            pltpu.VMEM((nch, _CH), jnp.int32),
            pltpu.VMEM((_CH, _D), jnp.float32),
            pltpu.VMEM((_CH, _D), jnp.float32),
            pltpu.VMEM((_CH, _D), jnp.float32),
            pltpu.VMEM((_CH, _D), jnp.float32),
            pltpu.VMEM((_CH, _D), jnp.float32),
            pltpu.VMEM((_CH, _D), jnp.float32),
            pltpu.SemaphoreType.DMA,
            pltpu.SemaphoreType.DMA,
            pltpu.SemaphoreType.DMA,
            pltpu.SemaphoreType.DMA,
            pltpu.SemaphoreType.DMA,
            pltpu.SemaphoreType.DMA,
        ],
    )
    def gather_mul(nf_hbm, src2_hbm, dst2_hbm, out_hbm,
                   sidx, didx, s0, s1, d0, d1, o0, o1,
                   ga0, ga1, gd0, gd1, wb0, wb1):
        srows = (s0, s1)
        drows = (d0, d1)
        orows = (o0, o1)
        gsem_s = (ga0, ga1)
        gsem_d = (gd0, gd1)
        wsem = (wb0, wb1)
        wid = lax.axis_index("s") * nc + lax.axis_index("c")
        ebase = wid * per_w

        def out_slice(i):
            return pl.ds(pl.multiple_of(ebase + i * _CH, 8), _CH)

        pltpu.sync_copy(src2_hbm.at[wid], sidx)
        pltpu.sync_copy(dst2_hbm.at[wid], didx)

        def start_gather(i, b):
            pltpu.async_copy(nf_hbm.at[sidx.at[i]], srows[b], gsem_s[b])
            pltpu.async_copy(nf_hbm.at[didx.at[i]], drows[b], gsem_d[b])

        def wait_gather(i, b):
            pltpu.make_async_copy(nf_hbm.at[sidx.at[i]], srows[b],
                                  gsem_s[b]).wait()
            pltpu.make_async_copy(nf_hbm.at[didx.at[i]], drows[b],
                                  gsem_d[b]).wait()

        def do_mul(b):
            sb, db, ob = srows[b], drows[b], orows[b]

            @plsc.parallel_loop(0, _CH, 1, unroll=2)
            def _(r):
                for j in range(_D // 16):
                    sl = pl.ds(j * 16, 16)
                    ob[r, sl] = sb[r, sl] * db[r, sl]

        def start_wb(i, b):
            pltpu.async_copy(orows[b], out_hbm.at[out_slice(i)], wsem[b])

        def wait_wb(i, b):
            pltpu.make_async_copy(orows[b], out_hbm.at[out_slice(i)],
                                  wsem[b]).wait()

        start_gather(0, 0)
        start_gather(1, 1)

        def pair(p, carry):
            i0 = p * 2
            for b in (0, 1):
                i = i0 + b
                wait_gather(i, b)

                @pl.when(p > 0)
                def _():
                    wait_wb(i - 2, b)

                do_mul(b)
                start_wb(i, b)

                @pl.when(i + 2 < nch)
                def _():
                    start_gather(i + 2, b)
            return carry

        lax.fori_loop(0, nch // 2, pair, 0)

        i = nch - 1
        wait_gather(i, 0)
        wait_wb(i - 2, 0)
        do_mul(0)
        start_wb(i, 0)
        wait_wb(nch - 2, 1)
        wait_wb(nch - 1, 0)

    return gather_mul


_TILE = 1280


def _edge_body(nj_ref, geot_ref, bid_ref, wxt_ref, wg_ref, bg_ref,
               g_ref, b_ref, out_ref):
    x = (jnp.dot(nj_ref[...], wxt_ref[...],
                 preferred_element_type=jnp.float32)
         + lax.dot_general(geot_ref[...], wg_ref[...],
                           (((0,), (0,)), ((), ())),
                           preferred_element_type=jnp.float32)
         + bg_ref[...])
    oht = (bid_ref[0] == lax.broadcasted_iota(jnp.int32, (_B, _TILE), 0)
           ).astype(jnp.float32)
    gamma = lax.dot_general(oht, g_ref[...], (((0,), (0,)), ((), ())),
                            preferred_element_type=jnp.float32)
    beta = lax.dot_general(oht, b_ref[...], (((0,), (0,)), ((), ())),
                           preferred_element_type=jnp.float32)
    out_ref[...] = jnp.maximum(x * gamma + beta, 0.0)


def _edge_stage(n_join, geot, bid3, wxt, wg, bg, gamma1, beta):
    grid = (_E // _TILE,)
    full = lambda i: (0, 0)
    return pl.pallas_call(
        _edge_body,
        grid=grid,
        in_specs=[
            pl.BlockSpec((_TILE, _D), lambda i: (i, 0)),
            pl.BlockSpec((_GEO, _TILE), lambda i: (0, i)),
            pl.BlockSpec((1, 1, _TILE), lambda i: (i, 0, 0)),
            pl.BlockSpec((_D, _D), full),
            pl.BlockSpec((_GEO, _D), full),
            pl.BlockSpec((1, _D), full),
            pl.BlockSpec((_B, _D), full),
            pl.BlockSpec((_B, _D), full),
        ],
        out_specs=pl.BlockSpec((_TILE, _D), lambda i: (i, 0)),
        out_shape=jax.ShapeDtypeStruct((_E, _D), jnp.float32),
        compiler_params=pltpu.CompilerParams(
            dimension_semantics=("arbitrary",)),
    )(n_join, geot, bid3, wxt, wg, bg, gamma1, beta)


_gather_mul = _make_gather_mul()


def kernel(node_feats, edge_index, edge_geo, cond_feats, batch_ids,
           W_nproj, b_nproj, W_egeo, b_egeo, W_cond, b_cond, W_x):
    nf, gamma1, beta, wg, bg = _prep(node_feats, W_nproj, b_nproj,
                                     cond_feats, W_cond, b_cond,
                                     W_egeo, b_egeo, W_x[_D:])
    src2 = edge_index[0].astype(jnp.int32).reshape(32, _E // (32 * _CH), _CH)
    dst2 = edge_index[1].astype(jnp.int32).reshape(32, _E // (32 * _CH), _CH)
    n_join = _gather_mul(nf, src2, dst2)
    bid3 = batch_ids.astype(jnp.int32).reshape(_E // _TILE, 1, _TILE)
    geot = edge_geo.T
    return _edge_stage(n_join, geot, bid3, W_x[:_D], wg, bg, gamma1, beta)


# speedup vs baseline: 7.6745x; 1.1945x over previous
import functools

import jax
import jax.numpy as jnp
from jax import lax
from jax.experimental import pallas as pl
from jax.experimental.pallas import tpu as pltpu
from jax.experimental.pallas import tpu_sc as plsc

_N = 10000
_E = 320000
_D = 128
_GEO = 16
_B = 16


def _prep_body(node_ref, wn_ref, bn_ref, cond_ref, wc_ref, bc_ref,
               weg_ref, beg_ref, wxb_ref,
               nf_ref, g_ref, b_ref, wg_ref, bg_ref):
    nf_ref[...] = (jnp.dot(node_ref[...], wn_ref[...],
                           preferred_element_type=jnp.float32) + bn_ref[...])
    gb = (jnp.dot(cond_ref[...], wc_ref[...],
                  preferred_element_type=jnp.float32) + bc_ref[...])
    g_ref[...] = gb[:, :_D] + 1.0
    b_ref[...] = gb[:, _D:]
    wg_ref[...] = jnp.dot(weg_ref[...], wxb_ref[...],
                          preferred_element_type=jnp.float32)
    bg_ref[...] = jnp.dot(beg_ref[...], wxb_ref[...],
                          preferred_element_type=jnp.float32)


def _prep(node_feats, W_nproj, b_nproj, cond_feats, W_cond, b_cond,
          W_egeo, b_egeo, wxb):
    return pl.pallas_call(
        _prep_body,
        out_shape=[
            jax.ShapeDtypeStruct((_N, _D), jnp.float32),
            jax.ShapeDtypeStruct((_B, _D), jnp.float32),
            jax.ShapeDtypeStruct((_B, _D), jnp.float32),
            jax.ShapeDtypeStruct((_GEO, _D), jnp.float32),
            jax.ShapeDtypeStruct((1, _D), jnp.float32),
        ],
    )(node_feats, W_nproj, b_nproj.reshape(1, _D),
      cond_feats, W_cond, b_cond.reshape(1, 2 * _D),
      W_egeo, b_egeo.reshape(1, _GEO), wxb)


_CH = 80


def _make_gather_mul():
    info = plsc.get_sparse_core_info()
    nc, ns = info.num_cores, info.num_subcores
    nw = nc * ns
    per_w = _E // nw
    nch = per_w // _CH

    mesh = plsc.VectorSubcoreMesh(core_axis_name="c", subcore_axis_name="s")

    @functools.partial(
        pl.kernel, mesh=mesh,
        out_type=jax.ShapeDtypeStruct((_E, _D), jnp.float32),
        scratch_types=[
            pltpu.VMEM((nch, _CH), jnp.int32),
            pltpu.VMEM((nch, _CH), jnp.int32),
            pltpu.VMEM((_CH, _D), jnp.float32),
            pltpu.VMEM((_CH, _D), jnp.float32),
            pltpu.VMEM((_CH, _D), jnp.float32),
            pltpu.VMEM((_CH, _D), jnp.float32),
            pltpu.VMEM((_CH, _D), jnp.float32),
            pltpu.VMEM((_CH, _D), jnp.float32),
            pltpu.SemaphoreType.DMA,
            pltpu.SemaphoreType.DMA,
            pltpu.SemaphoreType.DMA,
            pltpu.SemaphoreType.DMA,
            pltpu.SemaphoreType.DMA,
            pltpu.SemaphoreType.DMA,
        ],
    )
    def gather_mul(nf_hbm, src2_hbm, dst2_hbm, out_hbm,
                   sidx, didx, s0, s1, d0, d1, o0, o1,
                   ga0, ga1, gd0, gd1, wb0, wb1):
        srows = (s0, s1)
        drows = (d0, d1)
        orows = (o0, o1)
        gsem_s = (ga0, ga1)
        gsem_d = (gd0, gd1)
        wsem = (wb0, wb1)
        wid = lax.axis_index("s") * nc + lax.axis_index("c")
        ebase = wid * per_w

        def out_slice(i):
            return pl.ds(pl.multiple_of(ebase + i * _CH, 8), _CH)

        pltpu.sync_copy(src2_hbm.at[wid], sidx)
        pltpu.sync_copy(dst2_hbm.at[wid], didx)

        def start_gather(i, b):
            pltpu.async_copy(nf_hbm.at[sidx.at[i]], srows[b], gsem_s[b])
            pltpu.async_copy(nf_hbm.at[didx.at[i]], drows[b], gsem_d[b])

        def wait_gather(i, b):
            pltpu.make_async_copy(nf_hbm.at[sidx.at[i]], srows[b],
                                  gsem_s[b]).wait()
            pltpu.make_async_copy(nf_hbm.at[didx.at[i]], drows[b],
                                  gsem_d[b]).wait()

        def do_mul(b):
            sb, db, ob = srows[b], drows[b], orows[b]

            @plsc.parallel_loop(0, _CH, 1, unroll=2)
            def _(r):
                for j in range(_D // 16):
                    sl = pl.ds(j * 16, 16)
                    ob[r, sl] = sb[r, sl] * db[r, sl]

        def start_wb(i, b):
            pltpu.async_copy(orows[b], out_hbm.at[out_slice(i)], wsem[b])

        def wait_wb(i, b):
            pltpu.make_async_copy(orows[b], out_hbm.at[out_slice(i)],
                                  wsem[b]).wait()

        start_gather(0, 0)
        start_gather(1, 1)

        def pair(p, carry):
            i0 = p * 2
            for b in (0, 1):
                i = i0 + b
                wait_gather(i, b)

                @pl.when(p > 0)
                def _():
                    wait_wb(i - 2, b)

                do_mul(b)
                start_wb(i, b)

                @pl.when(i + 2 < nch)
                def _():
                    start_gather(i + 2, b)
            return carry

        lax.fori_loop(0, nch // 2, pair, 0)

        i = nch - 1
        wait_gather(i, 0)
        wait_wb(i - 2, 0)
        do_mul(0)
        start_wb(i, 0)
        wait_wb(nch - 2, 1)
        wait_wb(nch - 1, 0)

    return gather_mul


_TILE = 2560


def _edge_body(nj_ref, geot_ref, bid_ref, wxt_ref, wg_ref, bg_ref,
               g_ref, b_ref, out_ref):
    njb = nj_ref[...].astype(jnp.bfloat16)
    x = (jnp.dot(njb, wxt_ref[...],
                 preferred_element_type=jnp.float32)
         + lax.dot_general(geot_ref[...], wg_ref[...],
                           (((0,), (0,)), ((), ())),
                           preferred_element_type=jnp.float32)
         + bg_ref[...])
    oht = (bid_ref[0] == lax.broadcasted_iota(jnp.int32, (_B, _TILE), 0)
           ).astype(jnp.float32)
    gamma = lax.dot_general(oht, g_ref[...], (((0,), (0,)), ((), ())),
                            preferred_element_type=jnp.float32)
    beta = lax.dot_general(oht, b_ref[...], (((0,), (0,)), ((), ())),
                           preferred_element_type=jnp.float32)
    out_ref[...] = jnp.maximum(x * gamma + beta, 0.0)


def _edge_stage(n_join, geot, bid3, wxt, wg, bg, gamma1, beta):
    grid = (_E // _TILE,)
    full = lambda i: (0, 0)
    return pl.pallas_call(
        _edge_body,
        grid=grid,
        in_specs=[
            pl.BlockSpec((_TILE, _D), lambda i: (i, 0)),
            pl.BlockSpec((_GEO, _TILE), lambda i: (0, i)),
            pl.BlockSpec((1, 1, _TILE), lambda i: (i, 0, 0)),
            pl.BlockSpec((_D, _D), full),
            pl.BlockSpec((_GEO, _D), full),
            pl.BlockSpec((1, _D), full),
            pl.BlockSpec((_B, _D), full),
            pl.BlockSpec((_B, _D), full),
        ],
        out_specs=pl.BlockSpec((_TILE, _D), lambda i: (i, 0)),
        out_shape=jax.ShapeDtypeStruct((_E, _D), jnp.float32),
        compiler_params=pltpu.CompilerParams(
            dimension_semantics=("arbitrary",)),
    )(n_join, geot, bid3, wxt, wg, bg, gamma1, beta)


_gather_mul = _make_gather_mul()


def kernel(node_feats, edge_index, edge_geo, cond_feats, batch_ids,
           W_nproj, b_nproj, W_egeo, b_egeo, W_cond, b_cond, W_x):
    nf, gamma1, beta, wg, bg = _prep(node_feats, W_nproj, b_nproj,
                                     cond_feats, W_cond, b_cond,
                                     W_egeo, b_egeo, W_x[_D:])
    src2 = edge_index[0].astype(jnp.int32).reshape(32, _E // (32 * _CH), _CH)
    dst2 = edge_index[1].astype(jnp.int32).reshape(32, _E // (32 * _CH), _CH)
    n_join = _gather_mul(nf, src2, dst2)
    bid3 = batch_ids.astype(jnp.int32).reshape(_E // _TILE, 1, _TILE)
    geot = edge_geo.T
    wxtb = W_x[:_D].astype(jnp.bfloat16)
    return _edge_stage(n_join, geot, bid3, wxtb, wg, bg, gamma1, beta)


# speedup vs baseline: 8.5233x; 1.1106x over previous
import functools

import jax
import jax.numpy as jnp
from jax import lax
from jax.experimental import pallas as pl
from jax.experimental.pallas import tpu as pltpu
from jax.experimental.pallas import tpu_sc as plsc

_N = 10000
_E = 320000
_D = 128
_GEO = 16
_B = 16


def _prep_body(node_ref, wn_ref, bn_ref, cond_ref, wc_ref, bc_ref,
               weg_ref, beg_ref, wxb_ref,
               nf_ref, g_ref, b_ref, wg_ref, bg_ref):
    nf_ref[...] = (jnp.dot(node_ref[...], wn_ref[...],
                           preferred_element_type=jnp.float32) + bn_ref[...])
    gb = (jnp.dot(cond_ref[...], wc_ref[...],
                  preferred_element_type=jnp.float32) + bc_ref[...])
    g_ref[...] = gb[:, :_D] + 1.0
    b_ref[...] = gb[:, _D:]
    wg_ref[...] = jnp.dot(weg_ref[...], wxb_ref[...],
                          preferred_element_type=jnp.float32)
    bg_ref[...] = jnp.dot(beg_ref[...], wxb_ref[...],
                          preferred_element_type=jnp.float32)


def _prep(node_feats, W_nproj, b_nproj, cond_feats, W_cond, b_cond,
          W_egeo, b_egeo, wxb):
    return pl.pallas_call(
        _prep_body,
        out_shape=[
            jax.ShapeDtypeStruct((_N, _D), jnp.float32),
            jax.ShapeDtypeStruct((_B, _D), jnp.float32),
            jax.ShapeDtypeStruct((_B, _D), jnp.float32),
            jax.ShapeDtypeStruct((_GEO, _D), jnp.float32),
            jax.ShapeDtypeStruct((1, _D), jnp.float32),
        ],
    )(node_feats, W_nproj, b_nproj.reshape(1, _D),
      cond_feats, W_cond, b_cond.reshape(1, 2 * _D),
      W_egeo, b_egeo.reshape(1, _GEO), wxb)


_CH = 80


def _make_gather_mul():
    info = plsc.get_sparse_core_info()
    nc, ns = info.num_cores, info.num_subcores
    nw = nc * ns
    per_w = _E // nw
    nch = per_w // _CH

    mesh = plsc.VectorSubcoreMesh(core_axis_name="c", subcore_axis_name="s")

    @functools.partial(
        pl.kernel, mesh=mesh,
        compiler_params=pltpu.CompilerParams(needs_layout_passes=False),
        out_type=jax.ShapeDtypeStruct((_E // 2, _D), jnp.float32),
        scratch_types=[
            pltpu.VMEM((nch, _CH), jnp.int32),
            pltpu.VMEM((nch, _CH), jnp.int32),
            pltpu.VMEM((_CH, _D), jnp.float32),
            pltpu.VMEM((_CH, _D), jnp.float32),
            pltpu.VMEM((_CH, _D), jnp.float32),
            pltpu.VMEM((_CH, _D), jnp.float32),
            pltpu.VMEM((_CH // 2, _D), jnp.float32),
            pltpu.VMEM((_CH // 2, _D), jnp.float32),
            pltpu.SemaphoreType.DMA,
            pltpu.SemaphoreType.DMA,
            pltpu.SemaphoreType.DMA,
            pltpu.SemaphoreType.DMA,
            pltpu.SemaphoreType.DMA,
            pltpu.SemaphoreType.DMA,
        ],
    )
    def gather_mul(nf_hbm, src2_hbm, dst2_hbm, out_hbm,
                   sidx, didx, s0, s1, d0, d1, o0, o1,
                   ga0, ga1, gd0, gd1, wb0, wb1):
        srows = (s0, s1)
        drows = (d0, d1)
        orows = (o0, o1)
        gsem_s = (ga0, ga1)
        gsem_d = (gd0, gd1)
        wsem = (wb0, wb1)
        wid = lax.axis_index("s") * nc + lax.axis_index("c")
        ebase = wid * per_w

        def out_slice(i):
            return pl.ds(pl.multiple_of((ebase + i * _CH) // 2, 8), _CH // 2)

        pltpu.sync_copy(src2_hbm.at[wid], sidx)
        pltpu.sync_copy(dst2_hbm.at[wid], didx)

        def start_gather(i, b):
            pltpu.async_copy(nf_hbm.at[sidx.at[i]], srows[b], gsem_s[b])
            pltpu.async_copy(nf_hbm.at[didx.at[i]], drows[b], gsem_d[b])

        def wait_gather(i, b):
            pltpu.make_async_copy(nf_hbm.at[sidx.at[i]], srows[b],
                                  gsem_s[b]).wait()
            pltpu.make_async_copy(nf_hbm.at[didx.at[i]], drows[b],
                                  gsem_d[b]).wait()

        def do_mul(b):
            sb, db, ob = srows[b], drows[b], orows[b]

            @plsc.parallel_loop(0, _CH // 2, 1, unroll=2)
            def _(r2):
                for j in range(_D // 16):
                    sl = pl.ds(j * 16, 16)
                    p = sb[2 * r2, sl] * db[2 * r2, sl]
                    q = sb[2 * r2 + 1, sl] * db[2 * r2 + 1, sl]
                    packed = plsc.pack(p, q, format=plsc.PackFormat.INTERLEAVED)
                    ob[r2, sl] = plsc.bitcast(packed, jnp.float32)

        def start_wb(i, b):
            pltpu.async_copy(orows[b], out_hbm.at[out_slice(i)], wsem[b])

        def wait_wb(i, b):
            pltpu.make_async_copy(orows[b], out_hbm.at[out_slice(i)],
                                  wsem[b]).wait()

        start_gather(0, 0)
        start_gather(1, 1)

        def pair(p, carry):
            i0 = p * 2
            for b in (0, 1):
                i = i0 + b
                wait_gather(i, b)

                @pl.when(p > 0)
                def _():
                    wait_wb(i - 2, b)

                do_mul(b)
                start_wb(i, b)

                @pl.when(i + 2 < nch)
                def _():
                    start_gather(i + 2, b)
            return carry

        lax.fori_loop(0, nch // 2, pair, 0)

        i = nch - 1
        wait_gather(i, 0)
        wait_wb(i - 2, 0)
        do_mul(0)
        start_wb(i, 0)
        wait_wb(nch - 2, 1)
        wait_wb(nch - 1, 0)

    return gather_mul


_TILE = 2560


def _edge_body(nj_ref, geot_ref, bid_ref, wxt_ref, wg_ref, bg_ref,
               g_ref, b_ref, out_ref):
    njb = pltpu.bitcast(nj_ref[...], jnp.bfloat16)
    x = (jnp.dot(njb, wxt_ref[...],
                 preferred_element_type=jnp.float32)
         + lax.dot_general(geot_ref[...], wg_ref[...],
                           (((0,), (0,)), ((), ())),
                           preferred_element_type=jnp.float32)
         + bg_ref[...])
    oht = (bid_ref[0] == lax.broadcasted_iota(jnp.int32, (_B, _TILE), 0)
           ).astype(jnp.float32)
    gamma = lax.dot_general(oht, g_ref[...], (((0,), (0,)), ((), ())),
                            preferred_element_type=jnp.float32)
    beta = lax.dot_general(oht, b_ref[...], (((0,), (0,)), ((), ())),
                           preferred_element_type=jnp.float32)
    out_ref[...] = jnp.maximum(x * gamma + beta, 0.0)


def _edge_stage(n_join, geot, bid3, wxt, wg, bg, gamma1, beta):
    grid = (_E // _TILE,)
    full = lambda i: (0, 0)
    return pl.pallas_call(
        _edge_body,
        grid=grid,
        in_specs=[
            pl.BlockSpec((_TILE // 2, _D), lambda i: (i, 0)),
            pl.BlockSpec((_GEO, _TILE), lambda i: (0, i)),
            pl.BlockSpec((1, 1, _TILE), lambda i: (i, 0, 0)),
            pl.BlockSpec((_D, _D), full),
            pl.BlockSpec((_GEO, _D), full),
            pl.BlockSpec((1, _D), full),
            pl.BlockSpec((_B, _D), full),
            pl.BlockSpec((_B, _D), full),
        ],
        out_specs=pl.BlockSpec((_TILE, _D), lambda i: (i, 0)),
        out_shape=jax.ShapeDtypeStruct((_E, _D), jnp.float32),
        compiler_params=pltpu.CompilerParams(
            dimension_semantics=("arbitrary",)),
    )(n_join, geot, bid3, wxt, wg, bg, gamma1, beta)


_gather_mul = _make_gather_mul()


def kernel(node_feats, edge_index, edge_geo, cond_feats, batch_ids,
           W_nproj, b_nproj, W_egeo, b_egeo, W_cond, b_cond, W_x):
    nf, gamma1, beta, wg, bg = _prep(node_feats, W_nproj, b_nproj,
                                     cond_feats, W_cond, b_cond,
                                     W_egeo, b_egeo, W_x[_D:])
    src2 = edge_index[0].astype(jnp.int32).reshape(32, _E // (32 * _CH), _CH)
    dst2 = edge_index[1].astype(jnp.int32).reshape(32, _E // (32 * _CH), _CH)
    n_join = _gather_mul(nf, src2, dst2)
    bid3 = batch_ids.astype(jnp.int32).reshape(_E // _TILE, 1, _TILE)
    geot = edge_geo.T
    wxtb = W_x[:_D].astype(jnp.bfloat16)
    return _edge_stage(n_join, geot, bid3, wxtb, wg, bg, gamma1, beta)


# speedup vs baseline: 9.1218x; 1.0702x over previous
import functools

import jax
import jax.numpy as jnp
from jax import lax
from jax.experimental import pallas as pl
from jax.experimental.pallas import tpu as pltpu
from jax.experimental.pallas import tpu_sc as plsc

_N = 10000
_E = 320000
_D = 128
_GEO = 16
_B = 16


def _prep_body(node_ref, wn_ref, bn_ref, cond_ref, wc_ref, bc_ref,
               weg_ref, beg_ref, wxb_ref,
               nf_ref, g_ref, b_ref, wg_ref, bg_ref):
    nf_ref[...] = (jnp.dot(node_ref[...], wn_ref[...],
                           preferred_element_type=jnp.float32) + bn_ref[...])
    gb = (jnp.dot(cond_ref[...], wc_ref[...],
                  preferred_element_type=jnp.float32) + bc_ref[...])
    g_ref[...] = gb[:, :_D] + 1.0
    b_ref[...] = gb[:, _D:]
    wg_ref[...] = jnp.dot(weg_ref[...], wxb_ref[...],
                          preferred_element_type=jnp.float32)
    bg_ref[...] = jnp.dot(beg_ref[...], wxb_ref[...],
                          preferred_element_type=jnp.float32)


def _prep(node_feats, W_nproj, b_nproj, cond_feats, W_cond, b_cond,
          W_egeo, b_egeo, wxb):
    return pl.pallas_call(
        _prep_body,
        out_shape=[
            jax.ShapeDtypeStruct((_N, _D), jnp.float32),
            jax.ShapeDtypeStruct((_B, _D), jnp.float32),
            jax.ShapeDtypeStruct((_B, _D), jnp.float32),
            jax.ShapeDtypeStruct((_GEO, _D), jnp.float32),
            jax.ShapeDtypeStruct((1, _D), jnp.float32),
        ],
    )(node_feats, W_nproj, b_nproj.reshape(1, _D),
      cond_feats, W_cond, b_cond.reshape(1, 2 * _D),
      W_egeo, b_egeo.reshape(1, _GEO), wxb)


_CH = 80


def _make_gather_mul():
    info = plsc.get_sparse_core_info()
    nc, ns = info.num_cores, info.num_subcores
    nw = nc * ns
    per_w = _E // nw
    nch = per_w // _CH

    mesh = plsc.VectorSubcoreMesh(core_axis_name="c", subcore_axis_name="s")

    @functools.partial(
        pl.kernel, mesh=mesh,
        compiler_params=pltpu.CompilerParams(needs_layout_passes=False),
        out_type=jax.ShapeDtypeStruct((_E // 2, _D), jnp.float32),
        scratch_types=[
            pltpu.VMEM((nch, _CH), jnp.int32),
            pltpu.VMEM((nch, _CH), jnp.int32),
            pltpu.VMEM((_CH, _D), jnp.float32),
            pltpu.VMEM((_CH, _D), jnp.float32),
            pltpu.VMEM((_CH, _D), jnp.float32),
            pltpu.VMEM((_CH, _D), jnp.float32),
            pltpu.VMEM((_CH, _D), jnp.float32),
            pltpu.VMEM((_CH, _D), jnp.float32),
            pltpu.VMEM((_CH // 2, _D), jnp.float32),
            pltpu.VMEM((_CH // 2, _D), jnp.float32),
            pltpu.VMEM((_CH // 2, _D), jnp.float32),
            pltpu.SemaphoreType.DMA,
            pltpu.SemaphoreType.DMA,
            pltpu.SemaphoreType.DMA,
            pltpu.SemaphoreType.DMA,
            pltpu.SemaphoreType.DMA,
            pltpu.SemaphoreType.DMA,
            pltpu.SemaphoreType.DMA,
            pltpu.SemaphoreType.DMA,
            pltpu.SemaphoreType.DMA,
        ],
    )
    def gather_mul(nf_hbm, src2_hbm, dst2_hbm, out_hbm,
                   sidx, didx, s0, s1, s2, d0, d1, d2, o0, o1, o2,
                   ga0, ga1, ga2, gd0, gd1, gd2, wb0, wb1, wb2):
        srows = (s0, s1, s2)
        drows = (d0, d1, d2)
        orows = (o0, o1, o2)
        gsem_s = (ga0, ga1, ga2)
        gsem_d = (gd0, gd1, gd2)
        wsem = (wb0, wb1, wb2)
        wid = lax.axis_index("s") * nc + lax.axis_index("c")
        ebase = wid * per_w

        def out_slice(i):
            return pl.ds(pl.multiple_of((ebase + i * _CH) // 2, 8), _CH // 2)

        pltpu.sync_copy(src2_hbm.at[wid], sidx)
        pltpu.sync_copy(dst2_hbm.at[wid], didx)

        def start_gather(i, b):
            pltpu.async_copy(nf_hbm.at[sidx.at[i]], srows[b], gsem_s[b])
            pltpu.async_copy(nf_hbm.at[didx.at[i]], drows[b], gsem_d[b])

        def wait_gather(i, b):
            pltpu.make_async_copy(nf_hbm.at[sidx.at[i]], srows[b],
                                  gsem_s[b]).wait()
            pltpu.make_async_copy(nf_hbm.at[didx.at[i]], drows[b],
                                  gsem_d[b]).wait()

        def do_mul(b):
            sb, db, ob = srows[b], drows[b], orows[b]

            @plsc.parallel_loop(0, _CH // 2, 1, unroll=2)
            def _(r2):
                for j in range(_D // 16):
                    sl = pl.ds(j * 16, 16)
                    p = sb[2 * r2, sl] * db[2 * r2, sl]
                    q = sb[2 * r2 + 1, sl] * db[2 * r2 + 1, sl]
                    packed = plsc.pack(p, q, format=plsc.PackFormat.INTERLEAVED)
                    ob[r2, sl] = plsc.bitcast(packed, jnp.float32)

        def start_wb(i, b):
            pltpu.async_copy(orows[b], out_hbm.at[out_slice(i)], wsem[b])

        def wait_wb(i, b):
            pltpu.make_async_copy(orows[b], out_hbm.at[out_slice(i)],
                                  wsem[b]).wait()

        start_gather(0, 0)
        start_gather(1, 1)
        start_gather(2, 2)

        def triple(p, carry):
            i0 = p * 3
            for b in (0, 1, 2):
                i = i0 + b
                wait_gather(i, b)

                @pl.when(p > 0)
                def _():
                    wait_wb(i - 3, b)

                do_mul(b)
                start_wb(i, b)

                @pl.when(i + 3 < nch)
                def _():
                    start_gather(i + 3, b)
            return carry

        lax.fori_loop(0, nch // 3, triple, 0)

        for i, b in ((nch - 2, 0), (nch - 1, 1)):
            wait_gather(i, b)
            wait_wb(i - 3, b)
            do_mul(b)
            start_wb(i, b)
        wait_wb(nch - 3, 2)
        wait_wb(nch - 2, 0)
        wait_wb(nch - 1, 1)

    return gather_mul


_TILE = 3200


def _edge_body(nj_ref, geot_ref, bid_ref, wxt_ref, wg_ref, bg_ref,
               g_ref, b_ref, out_ref):
    njb = pltpu.bitcast(nj_ref[...], jnp.bfloat16)
    x = (jnp.dot(njb, wxt_ref[...],
                 preferred_element_type=jnp.float32)
         + lax.dot_general(geot_ref[...], wg_ref[...],
                           (((0,), (0,)), ((), ())),
                           preferred_element_type=jnp.float32)
         + bg_ref[...])
    oht = (bid_ref[0] == lax.broadcasted_iota(jnp.int32, (_B, _TILE), 0)
           ).astype(jnp.float32)
    gamma = lax.dot_general(oht, g_ref[...], (((0,), (0,)), ((), ())),
                            preferred_element_type=jnp.float32)
    beta = lax.dot_general(oht, b_ref[...], (((0,), (0,)), ((), ())),
                           preferred_element_type=jnp.float32)
    out_ref[...] = jnp.maximum(x * gamma + beta, 0.0)


def _edge_stage(n_join, geot, bid3, wxt, wg, bg, gamma1, beta):
    grid = (_E // _TILE,)
    full = lambda i: (0, 0)
    return pl.pallas_call(
        _edge_body,
        grid=grid,
        in_specs=[
            pl.BlockSpec((_TILE // 2, _D), lambda i: (i, 0)),
            pl.BlockSpec((_GEO, _TILE), lambda i: (0, i)),
            pl.BlockSpec((1, 1, _TILE), lambda i: (i, 0, 0)),
            pl.BlockSpec((_D, _D), full),
            pl.BlockSpec((_GEO, _D), full),
            pl.BlockSpec((1, _D), full),
            pl.BlockSpec((_B, _D), full),
            pl.BlockSpec((_B, _D), full),
        ],
        out_specs=pl.BlockSpec((_TILE, _D), lambda i: (i, 0)),
        out_shape=jax.ShapeDtypeStruct((_E, _D), jnp.float32),
        compiler_params=pltpu.CompilerParams(
            dimension_semantics=("arbitrary",)),
    )(n_join, geot, bid3, wxt, wg, bg, gamma1, beta)


_gather_mul = _make_gather_mul()


def kernel(node_feats, edge_index, edge_geo, cond_feats, batch_ids,
           W_nproj, b_nproj, W_egeo, b_egeo, W_cond, b_cond, W_x):
    nf, gamma1, beta, wg, bg = _prep(node_feats, W_nproj, b_nproj,
                                     cond_feats, W_cond, b_cond,
                                     W_egeo, b_egeo, W_x[_D:])
    src2 = edge_index[0].astype(jnp.int32).reshape(32, _E // (32 * _CH), _CH)
    dst2 = edge_index[1].astype(jnp.int32).reshape(32, _E // (32 * _CH), _CH)
    n_join = _gather_mul(nf, src2, dst2)
    bid3 = batch_ids.astype(jnp.int32).reshape(_E // _TILE, 1, _TILE)
    geot = edge_geo.T
    wxtb = W_x[:_D].astype(jnp.bfloat16)
    return _edge_stage(n_join, geot, bid3, wxtb, wg, bg, gamma1, beta)
